# baseline (speedup 1.0000x reference)
import functools

import jax
import jax.numpy as jnp
from jax import lax
from jax.experimental import pallas as pl
from jax.experimental.pallas import tpu as pltpu
from jax.experimental.pallas import tpu_sc as plsc

NUM_EXPERTS = 64
TOP_K = 8
HIDDEN = 4096
TOKENS = 32768
BT = 1024

NEG_INF = float("-inf")


def _logits_kernel(x_ref, w_ref, lt_ref):
    lt_ref[...] = jax.lax.dot_general(
        w_ref[...], x_ref[...],
        dimension_numbers=(((1,), (1,)), ((), ())),
        preferred_element_type=jnp.float32,
    )


def _logits_t(hidden_states, W):
    return pl.pallas_call(
        _logits_kernel,
        grid=(TOKENS // BT,),
        in_specs=[
            pl.BlockSpec((BT, HIDDEN), lambda i: (i, 0)),
            pl.BlockSpec((NUM_EXPERTS, HIDDEN), lambda i: (0, 0)),
        ],
        out_specs=pl.BlockSpec((NUM_EXPERTS, BT), lambda i: (0, i)),
        out_shape=jax.ShapeDtypeStruct((NUM_EXPERTS, TOKENS), jnp.float32),
        compiler_params=pltpu.CompilerParams(
            dimension_semantics=("arbitrary",),
        ),
    )(hidden_states, W)


_INFO = plsc.get_sparse_core_info()
_NW = _INFO.num_cores * _INFO.num_subcores
_TPW = TOKENS // _NW
_L = 16


def _sc_topk(lt_hbm, wt_hbm, it_hbm, buf, wbuf, ibuf):
    wid = lax.axis_index("s") * _INFO.num_cores + lax.axis_index("c")
    base = wid * _TPW
    pltpu.sync_copy(lt_hbm.at[:, pl.ds(base, _TPW)], buf)

    def group(g, carry):
        del carry
        col = g * _L
        vals = [buf[e, pl.ds(col, _L)] for e in range(NUM_EXPERTS)]
        idxs = [jnp.full((_L,), e, jnp.int32) for e in range(NUM_EXPERTS)]

        top_v = []
        top_i = []
        for _ in range(TOP_K):
            tv, ti = list(vals), list(idxs)
            while len(tv) > 1:
                nv, ni = [], []
                for j in range(0, len(tv), 2):
                    t = tv[j + 1] > tv[j]
                    nv.append(jnp.where(t, tv[j + 1], tv[j]))
                    ni.append(jnp.where(t, ti[j + 1], ti[j]))
                tv, ti = nv, ni
            m_v, m_i = tv[0], ti[0]
            top_v.append(m_v)
            top_i.append(m_i)
            vals = [
                jnp.where(m_i == idxs[e], NEG_INF, vals[e])
                for e in range(NUM_EXPERTS)
            ]

        es = [jnp.exp(v - top_v[0]) for v in top_v]
        tot = es[0]
        for e in es[1:]:
            tot = tot + e
        for k in range(TOP_K):
            wbuf[k, pl.ds(col, _L)] = es[k] / tot
            ibuf[k, pl.ds(col, _L)] = top_i[k]
        return 0

    lax.fori_loop(0, _TPW // _L, group, 0)
    pltpu.sync_copy(wbuf, wt_hbm.at[:, pl.ds(base, _TPW)])
    pltpu.sync_copy(ibuf, it_hbm.at[:, pl.ds(base, _TPW)])


@jax.jit
def kernel(hidden_states, W):
    lt = _logits_t(hidden_states, W)
    mesh = plsc.VectorSubcoreMesh(core_axis_name="c", subcore_axis_name="s")
    wt, it = functools.partial(
        pl.kernel,
        mesh=mesh,
        out_type=[
            jax.ShapeDtypeStruct((TOP_K, TOKENS), jnp.float32),
            jax.ShapeDtypeStruct((TOP_K, TOKENS), jnp.int32),
        ],
        scratch_types=[
            pltpu.VMEM((NUM_EXPERTS, _TPW), jnp.float32),
            pltpu.VMEM((TOP_K, _TPW), jnp.float32),
            pltpu.VMEM((TOP_K, _TPW), jnp.int32),
        ],
    )(_sc_topk)(lt)
    return (wt.T, it.T)


# speedup vs baseline: 1.5504x; 1.5504x over previous
import jax
import jax.numpy as jnp
from jax.experimental import pallas as pl
from jax.experimental.pallas import tpu as pltpu

NUM_EXPERTS = 64
TOP_K = 8
HIDDEN = 4096
TOKENS = 32768
BT = 1024
NH = 4

NEG_INF = float("-inf")


def _topk_softmax_t(lt):
    iota_f = jax.lax.broadcasted_iota(jnp.int32, lt.shape, 0).astype(jnp.float32)
    cur = lt
    vals = []
    idxs = []
    for _ in range(TOP_K):
        m = jnp.max(cur, axis=0, keepdims=True)
        is_max = cur == m
        idx = jnp.min(jnp.where(is_max, iota_f, 64.0), axis=0, keepdims=True)
        vals.append(m)
        idxs.append(idx)
        cur = jnp.where(iota_f == idx, NEG_INF, cur)

    v = jnp.concatenate(vals, axis=0)
    e = jnp.exp(v - v[0:1, :])
    w = e / jnp.sum(e, axis=0, keepdims=True)
    return w, jnp.concatenate(idxs, axis=0).astype(jnp.int32)


def _router_kernel(x_ref, w_ref, w_out_ref, i_out_ref):
    wmat = w_ref[...]
    rows = BT // NH
    lts = [
        jax.lax.dot_general(
            wmat, x_ref[pl.ds(h * rows, rows), :],
            dimension_numbers=(((1,), (1,)), ((), ())),
            preferred_element_type=jnp.float32,
        )
        for h in range(NH)
    ]
    for h in range(NH):
        w, i = _topk_softmax_t(lts[h])
        w_out_ref[:, pl.ds(h * rows, rows)] = w
        i_out_ref[:, pl.ds(h * rows, rows)] = i


@jax.jit
def kernel(hidden_states, W):
    grid = (TOKENS // BT,)
    out_w, out_i = pl.pallas_call(
        _router_kernel,
        grid=grid,
        in_specs=[
            pl.BlockSpec((BT, HIDDEN), lambda i: (i, 0)),
            pl.BlockSpec((NUM_EXPERTS, HIDDEN), lambda i: (0, 0)),
        ],
        out_specs=[
            pl.BlockSpec((TOP_K, BT), lambda i: (0, i)),
            pl.BlockSpec((TOP_K, BT), lambda i: (0, i)),
        ],
        out_shape=[
            jax.ShapeDtypeStruct((TOP_K, TOKENS), jnp.float32),
            jax.ShapeDtypeStruct((TOP_K, TOKENS), jnp.int32),
        ],
        compiler_params=pltpu.CompilerParams(
            dimension_semantics=("arbitrary",),
        ),
    )(hidden_states, W)
    return (out_w.T, out_i.T)
